# Initial kernel scaffold; baseline (speedup 1.0000x reference)
import jax
import jax.numpy as jnp
from jax.experimental import pallas as pl


def kernel(entity_ids, entity_types, jurisdictions, entity_table, type_table, juris_table):
    raise NotImplementedError("write your pallas kernel here")


# speedup vs baseline: 1.2444x; 1.2444x over previous
import functools

import jax
import jax.numpy as jnp
from jax import lax
from jax.experimental import pallas as pl
from jax.experimental.pallas import tpu as pltpu
from jax.experimental.pallas import tpu_sc as plsc

_HIDDEN = 8192
_LANES = 16
_NW = 32
_G = 4


def _combo_table(type_table, juris_table):
    n_types, hidden = type_table.shape
    n_juris = juris_table.shape[0]

    def body(t_ref, j_ref, o_ref):
        o_ref[...] = t_ref[...] + j_ref[...]

    out = pl.pallas_call(
        body,
        grid=(n_types * n_juris,),
        in_specs=[
            pl.BlockSpec((1, 1, hidden), lambda r: (r // n_juris, 0, 0)),
            pl.BlockSpec((1, 1, hidden), lambda r: (r % n_juris, 0, 0)),
        ],
        out_specs=pl.BlockSpec((1, 1, hidden), lambda r: (r, 0, 0)),
        out_shape=jax.ShapeDtypeStruct((n_types * n_juris, 1, hidden),
                                       jnp.float32),
    )(type_table.reshape(n_types, 1, hidden),
      juris_table.reshape(n_juris, 1, hidden))
    return out.reshape(n_types * n_juris, hidden)


def _make_sc_lookup(n_tokens):
    groups = n_tokens // (_NW * _G)
    mesh = plsc.VectorSubcoreMesh(core_axis_name="c", subcore_axis_name="s")
    nc = mesh.num_cores

    @functools.partial(
        pl.kernel,
        mesh=mesh,
        out_type=jax.ShapeDtypeStruct((n_tokens, _HIDDEN), jnp.float32),
        scratch_types=[
            pltpu.VMEM((groups, _G), jnp.int32),
            pltpu.VMEM((groups, _G), jnp.int32),
            pltpu.VMEM((_G, _HIDDEN), jnp.float32),
            pltpu.VMEM((_G, _HIDDEN), jnp.float32),
            pltpu.SemaphoreType.DMA,
            pltpu.SemaphoreType.DMA,
        ],
    )
    def lookup(ent_hbm, combo_hbm, eids_hbm, cids_hbm, out_hbm,
               eidx_v, cidx_v, ebuf, cbuf, sem_e, sem_c):
        wid = lax.axis_index("s") * nc + lax.axis_index("c")
        base = wid * (groups * _G)
        pltpu.sync_copy(eids_hbm.at[wid], eidx_v)
        pltpu.sync_copy(cids_hbm.at[wid], cidx_v)

        def g_body(g, carry):
            ce = pltpu.async_copy(ent_hbm.at[eidx_v.at[g]], ebuf, sem_e)
            cc = pltpu.async_copy(combo_hbm.at[cidx_v.at[g]], cbuf, sem_c)
            ce.wait()
            cc.wait()
            for r in range(_G):
                def add_body(i, c2, r=r):
                    off = i * _LANES
                    ebuf[r, pl.ds(off, _LANES)] = (
                        ebuf[r, pl.ds(off, _LANES)] + cbuf[r, pl.ds(off, _LANES)])
                    return c2
                lax.fori_loop(0, _HIDDEN // _LANES, add_body, 0)
            pltpu.sync_copy(ebuf, out_hbm.at[pl.ds(base + g * _G, _G)])
            return carry

        lax.fori_loop(0, groups, g_body, 0)

    return lookup


def kernel(entity_ids, entity_types, jurisdictions, entity_table,
           type_table, juris_table):
    b, l = entity_ids.shape
    n_tokens = b * l
    n_juris = juris_table.shape[0]
    groups = n_tokens // (_NW * _G)

    eids = entity_ids.reshape(_NW, groups, _G).astype(jnp.int32)
    cids = (entity_types * n_juris + jurisdictions).reshape(
        _NW, groups, _G).astype(jnp.int32)

    combo = _combo_table(type_table, juris_table)
    out = _make_sc_lookup(n_tokens)(entity_table, combo, eids, cids)
    return out.reshape(b, l, _HIDDEN)


# speedup vs baseline: 2.1949x; 1.7639x over previous
import functools

import jax
import jax.numpy as jnp
from jax import lax
from jax.experimental import pallas as pl
from jax.experimental.pallas import tpu as pltpu
from jax.experimental.pallas import tpu_sc as plsc

_HIDDEN = 8192
_LANES = 16
_NW = 32
_G = 2


def _combo_table(type_table, juris_table):
    n_types, hidden = type_table.shape
    n_juris = juris_table.shape[0]

    def body(t_ref, j_ref, o_ref):
        o_ref[...] = t_ref[...] + j_ref[...]

    out = pl.pallas_call(
        body,
        grid=(n_types * n_juris,),
        in_specs=[
            pl.BlockSpec((1, 1, hidden), lambda r: (r // n_juris, 0, 0)),
            pl.BlockSpec((1, 1, hidden), lambda r: (r % n_juris, 0, 0)),
        ],
        out_specs=pl.BlockSpec((1, 1, hidden), lambda r: (r, 0, 0)),
        out_shape=jax.ShapeDtypeStruct((n_types * n_juris, 1, hidden),
                                       jnp.float32),
    )(type_table.reshape(n_types, 1, hidden),
      juris_table.reshape(n_juris, 1, hidden))
    return out.reshape(n_types * n_juris, hidden)


_NBUF = 2
_UNROLL = 8


def _make_sc_lookup(n_tokens):
    groups = n_tokens // (_NW * _G)
    mesh = plsc.VectorSubcoreMesh(core_axis_name="c", subcore_axis_name="s")
    nc = mesh.num_cores

    @functools.partial(
        pl.kernel,
        mesh=mesh,
        out_type=jax.ShapeDtypeStruct((n_tokens, _HIDDEN), jnp.float32),
        scratch_types=[
            pltpu.VMEM((groups, 16), jnp.int32),
            [pltpu.VMEM((_G, _HIDDEN), jnp.float32)] * _NBUF,
            [pltpu.VMEM((_G, _HIDDEN), jnp.float32)] * _NBUF,
            [pltpu.SemaphoreType.DMA] * _NBUF,
            [pltpu.SemaphoreType.DMA] * _NBUF,
        ],
    )
    def lookup(ent_hbm, combo_hbm, ids_hbm, out_hbm,
               idx_v, ebufs, cbufs, esems, csems):
        wid = lax.axis_index("s") * nc + lax.axis_index("c")
        base = wid * (groups * _G)
        pltpu.sync_copy(ids_hbm.at[wid], idx_v)

        def gather_pair(g, b):
            pltpu.async_copy(
                ent_hbm.at[idx_v.at[g, pl.ds(0, _G)]], ebufs[b], esems[b])
            pltpu.async_copy(
                combo_hbm.at[idx_v.at[g, pl.ds(8, _G)]], cbufs[b], csems[b])

        def wait_pair(g, b):
            pltpu.make_async_copy(
                ent_hbm.at[idx_v.at[g, pl.ds(0, _G)]], ebufs[b],
                esems[b]).wait()
            pltpu.make_async_copy(
                combo_hbm.at[idx_v.at[g, pl.ds(8, _G)]], cbufs[b],
                csems[b]).wait()

        for b in range(_NBUF):
            gather_pair(b, b)

        def step(it, carry):
            for b in range(_NBUF):
                g = it * _NBUF + b
                wait_pair(g, b)
                eb, cb = ebufs[b], cbufs[b]
                for r in range(_G):
                    def add_body(i, c2, r=r, eb=eb, cb=cb):
                        for u in range(_UNROLL):
                            off = i * (_UNROLL * _LANES) + u * _LANES
                            eb[r, pl.ds(off, _LANES)] = (
                                eb[r, pl.ds(off, _LANES)]
                                + cb[r, pl.ds(off, _LANES)])
                        return c2
                    lax.fori_loop(0, _HIDDEN // (_UNROLL * _LANES),
                                  add_body, 0)
                pltpu.sync_copy(eb, out_hbm.at[pl.ds(base + g * _G, _G)])
                @pl.when(g + _NBUF < groups)
                def _issue(g=g, b=b):
                    gather_pair(g + _NBUF, b)
            return carry

        lax.fori_loop(0, groups // _NBUF, step, 0)

    return lookup


def kernel(entity_ids, entity_types, jurisdictions, entity_table,
           type_table, juris_table):
    b, l = entity_ids.shape
    n_tokens = b * l
    n_juris = juris_table.shape[0]
    groups = n_tokens // (_NW * _G)

    eids = entity_ids.reshape(_NW, groups, _G).astype(jnp.int32)
    cids = (entity_types * n_juris + jurisdictions).reshape(
        _NW, groups, _G).astype(jnp.int32)
    ids = jnp.zeros((_NW, groups, 16), jnp.int32)
    ids = ids.at[:, :, 0:_G].set(eids).at[:, :, 8:8 + _G].set(cids)

    combo = _combo_table(type_table, juris_table)
    out = _make_sc_lookup(n_tokens)(entity_table, combo, ids)
    return out.reshape(b, l, _HIDDEN)


# speedup vs baseline: 3.2019x; 1.4588x over previous
import functools

import jax
import jax.numpy as jnp
from jax import lax
from jax.experimental import pallas as pl
from jax.experimental.pallas import tpu as pltpu
from jax.experimental.pallas import tpu_sc as plsc

_HIDDEN = 8192
_LANES = 16
_NW = 32
_G = 2


def _combo_table(type_table, juris_table):
    n_types, hidden = type_table.shape
    n_juris = juris_table.shape[0]

    def body(t_ref, j_ref, o_ref):
        o_ref[...] = t_ref[...] + j_ref[...]

    out = pl.pallas_call(
        body,
        grid=(n_types * n_juris,),
        in_specs=[
            pl.BlockSpec((1, 1, hidden), lambda r: (r // n_juris, 0, 0)),
            pl.BlockSpec((1, 1, hidden), lambda r: (r % n_juris, 0, 0)),
        ],
        out_specs=pl.BlockSpec((1, 1, hidden), lambda r: (r, 0, 0)),
        out_shape=jax.ShapeDtypeStruct((n_types * n_juris, 1, hidden),
                                       jnp.float32),
    )(type_table.reshape(n_types, 1, hidden),
      juris_table.reshape(n_juris, 1, hidden))
    return out.reshape(n_types * n_juris, hidden)


_NBUF = 2
_UNROLL = 8


def _make_sc_lookup(n_b, n_l):
    n_tokens = n_b * n_l
    groups = n_tokens // (_NW * _G)
    mesh = plsc.VectorSubcoreMesh(core_axis_name="c", subcore_axis_name="s")
    nc = mesh.num_cores

    @functools.partial(
        pl.kernel,
        mesh=mesh,
        out_type=jax.ShapeDtypeStruct((n_b, n_l, _HIDDEN), jnp.float32),
        scratch_types=[
            pltpu.VMEM((groups, 16), jnp.int32),
            [pltpu.VMEM((_G, _HIDDEN), jnp.float32)] * _NBUF,
            [pltpu.VMEM((_G, _HIDDEN), jnp.float32)] * _NBUF,
            [pltpu.SemaphoreType.DMA] * _NBUF,
            [pltpu.SemaphoreType.DMA] * _NBUF,
        ],
    )
    def lookup(ent_hbm, combo_hbm, ids_hbm, out_hbm,
               idx_v, ebufs, cbufs, esems, csems):
        wid = lax.axis_index("s") * nc + lax.axis_index("c")
        base = wid * (groups * _G)
        pltpu.sync_copy(ids_hbm.at[wid], idx_v)

        def gather_pair(g, b):
            pltpu.async_copy(
                ent_hbm.at[idx_v.at[g, pl.ds(0, _G)]], ebufs[b], esems[b])
            pltpu.async_copy(
                combo_hbm.at[idx_v.at[g, pl.ds(8, _G)]], cbufs[b], csems[b])

        def wait_pair(g, b):
            pltpu.make_async_copy(
                ent_hbm.at[idx_v.at[g, pl.ds(0, _G)]], ebufs[b],
                esems[b]).wait()
            pltpu.make_async_copy(
                combo_hbm.at[idx_v.at[g, pl.ds(8, _G)]], cbufs[b],
                csems[b]).wait()

        for b in range(_NBUF):
            gather_pair(b, b)

        def step(it, carry):
            for b in range(_NBUF):
                g = it * _NBUF + b
                wait_pair(g, b)
                eb, cb = ebufs[b], cbufs[b]
                for r in range(_G):
                    def add_body(i, c2, r=r, eb=eb, cb=cb):
                        for u in range(_UNROLL):
                            off = i * (_UNROLL * _LANES) + u * _LANES
                            eb[r, pl.ds(off, _LANES)] = (
                                eb[r, pl.ds(off, _LANES)]
                                + cb[r, pl.ds(off, _LANES)])
                        return c2
                    lax.fori_loop(0, _HIDDEN // (_UNROLL * _LANES),
                                  add_body, 0)
                t = base + g * _G
                bi = t // n_l
                li = t - bi * n_l
                pltpu.sync_copy(eb, out_hbm.at[bi, pl.ds(li, _G)])
                @pl.when(g + _NBUF < groups)
                def _issue(g=g, b=b):
                    gather_pair(g + _NBUF, b)
            return carry

        lax.fori_loop(0, groups // _NBUF, step, 0)

    return lookup


def kernel(entity_ids, entity_types, jurisdictions, entity_table,
           type_table, juris_table):
    b, l = entity_ids.shape
    n_tokens = b * l
    n_juris = juris_table.shape[0]
    groups = n_tokens // (_NW * _G)

    eids = entity_ids.reshape(_NW, groups, _G).astype(jnp.int32)
    cids = (entity_types * n_juris + jurisdictions).reshape(
        _NW, groups, _G).astype(jnp.int32)
    ids = jnp.zeros((_NW, groups, 16), jnp.int32)
    ids = ids.at[:, :, 0:_G].set(eids).at[:, :, 8:8 + _G].set(cids)

    combo = _combo_table(type_table, juris_table)
    return _make_sc_lookup(b, l)(entity_table, combo, ids)


# speedup vs baseline: 5.1823x; 1.6185x over previous
import functools

import jax
import jax.numpy as jnp
from jax import lax
from jax.experimental import pallas as pl
from jax.experimental.pallas import tpu as pltpu
from jax.experimental.pallas import tpu_sc as plsc

_HIDDEN = 8192
_LANES = 16
_NW = 32
_G = 2


def _combo_table(type_table, juris_table):
    n_types, hidden = type_table.shape
    n_juris = juris_table.shape[0]

    def body(t_ref, j_ref, o_ref):
        o_ref[...] = t_ref[...] + j_ref[...]

    out = pl.pallas_call(
        body,
        grid=(n_types * n_juris,),
        in_specs=[
            pl.BlockSpec((1, 1, hidden), lambda r: (r // n_juris, 0, 0)),
            pl.BlockSpec((1, 1, hidden), lambda r: (r % n_juris, 0, 0)),
        ],
        out_specs=pl.BlockSpec((1, 1, hidden), lambda r: (r, 0, 0)),
        out_shape=jax.ShapeDtypeStruct((n_types * n_juris, 1, hidden),
                                       jnp.float32),
    )(type_table.reshape(n_types, 1, hidden),
      juris_table.reshape(n_juris, 1, hidden))
    return out.reshape(n_types * n_juris, hidden)


_NBUF = 2
_UNROLL = 8


def _make_sc_lookup(n_b, n_l):
    n_tokens = n_b * n_l
    wb = n_b // _NW
    bgroups = wb // _G
    groups = n_tokens // (_NW * _G)
    mesh = plsc.VectorSubcoreMesh(core_axis_name="c", subcore_axis_name="s")
    nc = mesh.num_cores

    @functools.partial(
        pl.kernel,
        mesh=mesh,
        out_type=jax.ShapeDtypeStruct((n_l, n_b, _HIDDEN), jnp.float32),
        scratch_types=[
            pltpu.VMEM((groups, 16), jnp.int32),
            [pltpu.VMEM((_G, _HIDDEN), jnp.float32)] * _NBUF,
            [pltpu.VMEM((_G, _HIDDEN), jnp.float32)] * _NBUF,
            [pltpu.SemaphoreType.DMA] * _NBUF,
            [pltpu.SemaphoreType.DMA] * _NBUF,
        ],
    )
    def lookup(ent_hbm, combo_hbm, ids_hbm, out_hbm,
               idx_v, ebufs, cbufs, esems, csems):
        wid = lax.axis_index("s") * nc + lax.axis_index("c")
        base_b = wid * wb
        pltpu.sync_copy(ids_hbm.at[wid], idx_v)

        def gather_pair(g, b):
            pltpu.async_copy(
                ent_hbm.at[idx_v.at[g, pl.ds(0, _G)]], ebufs[b], esems[b])
            pltpu.async_copy(
                combo_hbm.at[idx_v.at[g, pl.ds(8, _G)]], cbufs[b], csems[b])

        def wait_pair(g, b):
            pltpu.make_async_copy(
                ent_hbm.at[idx_v.at[g, pl.ds(0, _G)]], ebufs[b],
                esems[b]).wait()
            pltpu.make_async_copy(
                combo_hbm.at[idx_v.at[g, pl.ds(8, _G)]], cbufs[b],
                csems[b]).wait()

        for b in range(_NBUF):
            gather_pair(b, b)

        def step(it, carry):
            for b in range(_NBUF):
                g = it * _NBUF + b
                wait_pair(g, b)
                eb, cb = ebufs[b], cbufs[b]
                for r in range(_G):
                    def add_body(i, c2, r=r, eb=eb, cb=cb):
                        for u in range(_UNROLL):
                            off = i * (_UNROLL * _LANES) + u * _LANES
                            eb[r, pl.ds(off, _LANES)] = (
                                eb[r, pl.ds(off, _LANES)]
                                + cb[r, pl.ds(off, _LANES)])
                        return c2
                    lax.fori_loop(0, _HIDDEN // (_UNROLL * _LANES),
                                  add_body, 0)
                li = g // bgroups
                bi = base_b + (g - li * bgroups) * _G
                pltpu.sync_copy(eb, out_hbm.at[li, pl.ds(bi, _G)])
                @pl.when(g + _NBUF < groups)
                def _issue(g=g, b=b):
                    gather_pair(g + _NBUF, b)
            return carry

        lax.fori_loop(0, groups // _NBUF, step, 0)

    return lookup


def kernel(entity_ids, entity_types, jurisdictions, entity_table,
           type_table, juris_table):
    b, l = entity_ids.shape
    n_tokens = b * l
    n_juris = juris_table.shape[0]
    wb = b // _NW
    groups = n_tokens // (_NW * _G)

    def order(a):
        return (a.astype(jnp.int32).reshape(_NW, wb, l)
                .transpose(0, 2, 1).reshape(_NW, groups, _G))

    eids = order(entity_ids)
    cids = order(entity_types * n_juris + jurisdictions)
    zpad = jnp.zeros((_NW, groups, 8 - _G), jnp.int32)
    ids = jnp.concatenate([eids, zpad, cids, zpad], axis=2)

    combo = _combo_table(type_table, juris_table)
    out_lm = _make_sc_lookup(b, l)(entity_table, combo, ids)
    return jnp.transpose(out_lm, (1, 0, 2))
